# Initial kernel scaffold; baseline (speedup 1.0000x reference)
import jax
import jax.numpy as jnp
from jax.experimental import pallas as pl


def kernel(in_item_id, table, W0, b0, W1, b1, W2, b2, W_out, b_out):
    raise NotImplementedError("write your pallas kernel here")


# speedup vs baseline: 1.9242x; 1.9242x over previous
import functools

import jax
import jax.numpy as jnp
from jax import lax
from jax.experimental import pallas as pl
from jax.experimental.pallas import tpu as pltpu
from jax.experimental.pallas import tpu_sc as plsc

B = 4096
L = 200
D = 128
L_HALF = 104
L_PAD = 2 * L_HALF

NC, NS, LANES = 2, 16, 16
NW = NC * NS
BPW = B // NW
NVR = D // LANES

_sc_mesh = plsc.VectorSubcoreMesh(core_axis_name="c", subcore_axis_name="s")


@functools.partial(
    pl.kernel,
    out_type=jax.ShapeDtypeStruct((B, D), jnp.float32),
    mesh=_sc_mesh,
    scratch_types=[
        pltpu.VMEM((BPW, 2, L_HALF), jnp.int32),
        pltpu.VMEM((2, 2, L_HALF, D), jnp.float32),
        pltpu.VMEM((BPW, D), jnp.float32),
        pltpu.SemaphoreType.DMA,
        pltpu.SemaphoreType.DMA,
    ],
)
def _sc_pool(ids_hbm, table_hbm, out_hbm, ids_v, rows_v, out_v, sem0, sem1):
    wid = lax.axis_index("s") * NC + lax.axis_index("c")
    base = wid * BPW
    pltpu.sync_copy(ids_hbm.at[pl.ds(base, BPW)], ids_v)

    sems = (sem0, sem1)

    def start_gather(b, slot):
        pltpu.async_copy(table_hbm.at[ids_v.at[b, 0]], rows_v.at[slot, 0], sems[slot])
        pltpu.async_copy(table_hbm.at[ids_v.at[b, 1]], rows_v.at[slot, 1], sems[slot])

    def wait_gather(slot):
        pltpu.make_async_copy(table_hbm.at[ids_v.at[0, 0]], rows_v.at[slot, 0], sems[slot]).wait()
        pltpu.make_async_copy(table_hbm.at[ids_v.at[0, 1]], rows_v.at[slot, 1], sems[slot]).wait()

    def reduce_rows(b, slot):
        buf = rows_v.at[slot]
        def body(l, accs):
            out = []
            for h in range(2):
                for j in range(NVR):
                    out.append(accs[h * NVR + j] + buf[h, l, pl.ds(j * LANES, LANES)])
            return tuple(out)
        accs = lax.fori_loop(
            0, L_HALF, body,
            tuple(jnp.zeros((LANES,), jnp.float32) for _ in range(2 * NVR)),
        )
        for j in range(NVR):
            out_v[b, pl.ds(j * LANES, LANES)] = accs[j] + accs[NVR + j]

    start_gather(0, 0)

    def pair_body(p, _):
        b0 = p * 2
        start_gather(b0 + 1, 1)
        wait_gather(0)
        reduce_rows(b0, 0)

        @pl.when(b0 + 2 < BPW)
        def _():
            start_gather(b0 + 2, 0)
        wait_gather(1)
        reduce_rows(b0 + 1, 1)
        return _

    lax.fori_loop(0, BPW // 2, pair_body, None)
    pltpu.sync_copy(out_v, out_hbm.at[pl.ds(base, BPW)])


_TC_CHUNK = 512


def _mlp_body(ids_ref, x_ref, w0_ref, b0_ref, w1_ref, b1_ref, w2_ref, b2_ref,
              wo_ref, bo_ref, o_ref):
    cnt = jnp.sum((ids_ref[...] != 0).astype(jnp.float32), axis=1, keepdims=True)
    x = x_ref[...] * lax.rsqrt(cnt)
    h = jnp.maximum(jnp.dot(x, w0_ref[...], preferred_element_type=jnp.float32)
                    + b0_ref[...], 0.0)
    h = jnp.maximum(jnp.dot(h, w1_ref[...], preferred_element_type=jnp.float32)
                    + b1_ref[...], 0.0)
    h = jnp.maximum(jnp.dot(h, w2_ref[...], preferred_element_type=jnp.float32)
                    + b2_ref[...], 0.0)
    o_ref[...] = (jnp.dot(h, wo_ref[...], preferred_element_type=jnp.float32)
                  + bo_ref[...])


def _resident(shape):
    return pl.BlockSpec(shape, lambda i: (0,) * len(shape))


_mlp_call = pl.pallas_call(
    _mlp_body,
    grid=(B // _TC_CHUNK,),
    in_specs=[
        pl.BlockSpec((_TC_CHUNK, L), lambda i: (i, 0)),
        pl.BlockSpec((_TC_CHUNK, D), lambda i: (i, 0)),
        _resident((D, 2048)),
        _resident((1, 2048)),
        _resident((2048, 1024)),
        _resident((1, 1024)),
        _resident((1024, 2048)),
        _resident((1, 2048)),
        _resident((2048, D)),
        _resident((1, D)),
    ],
    out_specs=pl.BlockSpec((_TC_CHUNK, D), lambda i: (i, 0)),
    out_shape=jax.ShapeDtypeStruct((B, D), jnp.float32),
)


def kernel(in_item_id, table, W0, b0, W1, b1, W2, b2, W_out, b_out):
    ids = in_item_id.astype(jnp.int32)
    ids_pad = jnp.pad(ids, ((0, 0), (0, L_PAD - L))).reshape(B, 2, L_HALF)
    pooled = _sc_pool(ids_pad, table)
    return _mlp_call(ids, pooled, W0, b0.reshape(1, -1), W1, b1.reshape(1, -1),
                     W2, b2.reshape(1, -1), W_out, b_out.reshape(1, -1))
